# baseline (speedup 1.0000x reference)
import jax
import jax.numpy as jnp
from jax.experimental import pallas as pl
from jax.experimental.pallas import tpu as pltpu

N = 10000
NFEAT = 128
NHID = 16
BM = 400
NB = N // BM


def _gcn_body(x_ref, adj_ref, W1_ref, b1_ref, W2_ref, b2_ref, w3t_ref, b3_ref,
              out_ref, s1_ref, g_ref, mx_ref):
    p = pl.program_id(0)
    i = pl.program_id(1)

    @pl.when((p == 0) & (i == 0))
    def _init():
        s1_ref[...] = jnp.dot(x_ref[...], W1_ref[...],
                              preferred_element_type=jnp.float32)

    @pl.when(p == 0)
    def _pass1():
        h = jnp.dot(adj_ref[...], s1_ref[...],
                    preferred_element_type=jnp.float32)
        h = jnp.maximum(h + b1_ref[...], 0.0)
        g_ref[pl.ds(i * BM, BM), :] = jnp.dot(
            h, W2_ref[...], preferred_element_type=jnp.float32)

    @pl.when(p == 1)
    def _pass2():
        y = jnp.dot(adj_ref[...], g_ref[...],
                    preferred_element_type=jnp.float32)
        m = jnp.max(y, axis=0, keepdims=True)

        @pl.when(i == 0)
        def _():
            mx_ref[...] = m

        @pl.when(i > 0)
        def _():
            mx_ref[...] = jnp.maximum(mx_ref[...], m)

        @pl.when(i == NB - 1)
        def _finish():
            mm = mx_ref[...] + b2_ref[...]
            o = jnp.sum(mm * w3t_ref[...], axis=1, keepdims=True) + b3_ref[...]
            out_ref[0, :, :] = o


def kernel(x, adj, W1, b1, W2, b2, w3, b3):
    b1r = b1.reshape(1, NHID)
    b2r = b2.reshape(1, 2)
    w3t = w3.reshape(1, 2)
    b3r = b3.reshape(1, 1)

    out = pl.pallas_call(
        _gcn_body,
        grid=(2, NB),
        in_specs=[
            pl.BlockSpec((N, NFEAT), lambda p, i: (0, 0)),
            pl.BlockSpec((BM, N), lambda p, i: (i, 0)),
            pl.BlockSpec((NFEAT, NHID), lambda p, i: (0, 0)),
            pl.BlockSpec((1, NHID), lambda p, i: (0, 0)),
            pl.BlockSpec((NHID, 2), lambda p, i: (0, 0)),
            pl.BlockSpec((1, 2), lambda p, i: (0, 0)),
            pl.BlockSpec((1, 2), lambda p, i: (0, 0)),
            pl.BlockSpec((1, 1), lambda p, i: (0, 0)),
        ],
        out_specs=pl.BlockSpec((1, 1, 1), lambda p, i: (0, 0, 0)),
        out_shape=jax.ShapeDtypeStruct((1, 1, 1), jnp.float32),
        scratch_shapes=[
            pltpu.VMEM((N, NHID), jnp.float32),
            pltpu.VMEM((N, 2), jnp.float32),
            pltpu.VMEM((1, 2), jnp.float32),
        ],
        compiler_params=pltpu.CompilerParams(
            dimension_semantics=("arbitrary", "arbitrary"),
        ),
    )(x, adj, W1, b1r, W2, b2r, w3t, b3r)
    return out
